# Initial kernel scaffold; baseline (speedup 1.0000x reference)
import jax
import jax.numpy as jnp
from jax.experimental import pallas as pl


def kernel(x, edge_index, edge_attr, W1, b1, W2, b2, W_mu, b_mu, W_std, b_std):
    raise NotImplementedError("write your pallas kernel here")


# speedup vs baseline: 2.5822x; 2.5822x over previous
import functools

import jax
import jax.numpy as jnp
from jax import lax
from jax.experimental import pallas as pl
from jax.experimental.pallas import tpu as pltpu
from jax.experimental.pallas import tpu_sc as plsc

N = 10000
NP = 10240
D_IN = 128
H1 = 256
H2 = 128

NC = 2
NS = 16
NW = NC * NS
RPT = NP // NS

E2 = 320000 + N
EP = 360448
EROWS = EP // 128
DROWS = EROWS // NW
EK = 256

_mesh = plsc.VectorSubcoreMesh(core_axis_name="c", subcore_axis_name="s")


@functools.partial(
    pl.kernel,
    out_type=jax.ShapeDtypeStruct((NC, NP), jnp.float32),
    mesh=_mesh,
    scratch_types=[
        pltpu.VMEM((DROWS, 128), jnp.int32),
        pltpu.VMEM((DROWS, 128), jnp.float32),
        pltpu.VMEM((RPT,), jnp.float32),
        pltpu.VMEM_SHARED((NP,), jnp.float32),
    ],
)
def _deg_kernel(dst2, ew2, degp, didx, ewv, zb, acc):
    c = lax.axis_index("c")
    s = lax.axis_index("s")
    w = c * NS + s
    def zb_body(r, carry):
        zb[pl.ds(r * 16, 16)] = jnp.zeros((16,), jnp.float32)
        return carry
    lax.fori_loop(0, RPT // 16, zb_body, 0)
    pltpu.sync_copy(zb, acc.at[pl.ds(s * RPT, RPT)])
    plsc.subcore_barrier()
    pltpu.sync_copy(dst2.at[pl.ds(w * DROWS, DROWS)], didx)
    pltpu.sync_copy(ew2.at[pl.ds(w * DROWS, DROWS)], ewv)
    def sc_body(j, carry):
        pltpu.sync_copy(ewv.at[j], acc.at[didx.at[j]], add=True)
        return carry
    lax.fori_loop(0, DROWS, sc_body, 0)
    plsc.subcore_barrier()
    pltpu.sync_copy(acc.at[pl.ds(s * RPT, RPT)], degp.at[c].at[pl.ds(s * RPT, RPT)])


SCE = 2048
SROWS = SCE // 128


def _make_agg(ept, split_cores):
    irt = ept // 128
    nsc = ept // SCE

    @functools.partial(
        pl.kernel,
        out_type=jax.ShapeDtypeStruct((NC, NP, 128), jnp.float32),
        mesh=_mesh,
        scratch_types=[
            pltpu.VMEM((SROWS, 128), jnp.int32),
            pltpu.VMEM((SROWS, 128), jnp.int32),
            pltpu.VMEM((SROWS, 128), jnp.float32),
            pltpu.VMEM((EK, 128), jnp.float32),
            pltpu.VMEM((32, 128), jnp.float32),
            pltpu.VMEM_SHARED((NP, 128), jnp.float32),
            pltpu.SemaphoreType.DMA,
        ],
    )
    def agg(tbl, srcx, dstx, ewf, out, sidx, didx, ewv, rows, zbuf, acc, sem):
        c = lax.axis_index("c")
        s = lax.axis_index("s")
        w = c * NS + s if split_cores else s
        row0 = s * RPT
        def zfill(r, carry):
            for g in range(8):
                zbuf[r, pl.ds(g * 16, 16)] = jnp.zeros((16,), jnp.float32)
            return carry
        lax.fori_loop(0, 32, zfill, 0)
        def zcp(k, carry):
            pltpu.sync_copy(zbuf, acc.at[pl.ds(row0 + k * 32, 32)])
            return carry
        lax.fori_loop(0, RPT // 32, zcp, 0)
        plsc.subcore_barrier()

        def superchunk(t, carry):
            base = w * irt + t * SROWS
            pltpu.sync_copy(srcx.at[pl.ds(base, SROWS)], sidx)
            pltpu.sync_copy(dstx.at[pl.ds(base, SROWS)], didx)
            pltpu.sync_copy(ewf.at[pl.ds(base, SROWS)], ewv)

            def chunk(q, carry2):
                jj = q * (EK // 128)
                cps = [
                    pltpu.async_copy(
                        tbl.at[c].at[sidx.at[jj + j]],
                        rows.at[pl.ds(j * 128, 128)],
                        sem,
                    )
                    for j in range(EK // 128)
                ]
                for cp in cps:
                    cp.wait()
                def scale(k, inner):
                    r0 = k * 16
                    wvec = ewv[jj + r0 // 128, pl.ds(r0 % 128, 16)]
                    for i in range(16):
                        r = r0 + i
                        wv = wvec[i]
                        for g in range(8):
                            rows[r, pl.ds(g * 16, 16)] = (
                                rows[r, pl.ds(g * 16, 16)] * wv)
                    return inner
                lax.fori_loop(0, EK // 16, scale, 0)
                for j in range(EK // 128):
                    pltpu.sync_copy(
                        rows.at[pl.ds(j * 128, 128)],
                        acc.at[didx.at[jj + j]],
                        add=True,
                    )
                return carry2
            lax.fori_loop(0, SCE // EK, chunk, 0)
            return carry
        lax.fori_loop(0, nsc, superchunk, 0)
        plsc.subcore_barrier()
        pltpu.sync_copy(
            acc.at[pl.ds(row0, RPT)],
            out.at[c].at[pl.ds(row0, RPT)],
        )
    return agg


_agg = _make_agg(EP // NS, False)


_BR = 1024


def _t1_body(degp_ref, x_ref, dinv_ref, t0_ref):
    deg = degp_ref[0] + degp_ref[1]
    dinv = jnp.where(deg > 0, lax.rsqrt(jnp.maximum(deg, 1e-12)), 0.0)
    dinv_ref[...] = dinv
    t = x_ref[...] * dinv[:, None]
    t0_ref[0] = t
    t0_ref[1] = t


def _t1(degp, x):
    return pl.pallas_call(
        _t1_body,
        grid=(NP // _BR,),
        in_specs=[
            pl.BlockSpec((NC, _BR), lambda i: (0, i)),
            pl.BlockSpec((_BR, D_IN), lambda i: (i, 0)),
        ],
        out_specs=[
            pl.BlockSpec((_BR,), lambda i: (i,)),
            pl.BlockSpec((NC, _BR, 128), lambda i: (0, i, 0)),
        ],
        out_shape=[
            jax.ShapeDtypeStruct((NP,), jnp.float32),
            jax.ShapeDtypeStruct((NC, NP, 128), jnp.float32),
        ],
    )(degp, x)


def _mid1_body(a_ref, dinv_ref, w_ref, b_ref, t_ref):
    dinv = dinv_ref[...]
    af = a_ref[0] * dinv[:, None]
    h = jnp.dot(af, w_ref[...], preferred_element_type=jnp.float32)
    h = jnp.maximum(h + b_ref[...][None, :], 0.0) * dinv[:, None]
    t_ref[0] = h[:, :128]
    t_ref[1] = h[:, 128:]


def _mid1(a, dinv, w, b):
    return pl.pallas_call(
        _mid1_body,
        grid=(NP // _BR,),
        in_specs=[
            pl.BlockSpec((NC, _BR, 128), lambda i: (0, i, 0)),
            pl.BlockSpec((_BR,), lambda i: (i,)),
            pl.BlockSpec((D_IN, H1), lambda i: (0, 0)),
            pl.BlockSpec((H1,), lambda i: (0,)),
        ],
        out_specs=pl.BlockSpec((NC, _BR, 128), lambda i: (0, i, 0)),
        out_shape=jax.ShapeDtypeStruct((NC, NP, 128), jnp.float32),
    )(a, dinv, w, b)


def _mid2_body(a_ref, dinv_ref, w_ref, b_ref, t_ref):
    dinv = dinv_ref[...]
    af = jnp.concatenate([a_ref[0], a_ref[1]], axis=1) * dinv[:, None]
    h = jnp.dot(af, w_ref[...], preferred_element_type=jnp.float32)
    h = jnp.maximum(h + b_ref[...][None, :], 0.0) * dinv[:, None]
    t_ref[0] = h[:, :128]
    t_ref[1] = h[:, 128:]


def _mid2(a, dinv, w, b):
    return pl.pallas_call(
        _mid2_body,
        grid=(NP // _BR,),
        in_specs=[
            pl.BlockSpec((NC, _BR, 128), lambda i: (0, i, 0)),
            pl.BlockSpec((_BR,), lambda i: (i,)),
            pl.BlockSpec((H1, H1), lambda i: (0, 0)),
            pl.BlockSpec((H1,), lambda i: (0,)),
        ],
        out_specs=pl.BlockSpec((NC, _BR, 128), lambda i: (0, i, 0)),
        out_shape=jax.ShapeDtypeStruct((NC, NP, 128), jnp.float32),
    )(a, dinv, w, b)


def _head_body(a_ref, dinv_ref, wm_ref, bm_ref, ws_ref, bs_ref, mu_ref, std_ref):
    dinv = dinv_ref[...]
    g = jnp.concatenate([a_ref[0], a_ref[1]], axis=1) * dinv[:, None]
    mu_ref[...] = jnp.dot(g, wm_ref[...], preferred_element_type=jnp.float32) \
        + bm_ref[...][None, :]
    std_ref[...] = jnp.dot(g, ws_ref[...], preferred_element_type=jnp.float32) \
        + bs_ref[...][None, :]


def _head(a, dinv, wm, bm, ws, bs):
    return pl.pallas_call(
        _head_body,
        grid=(NP // _BR,),
        in_specs=[
            pl.BlockSpec((NC, _BR, 128), lambda i: (0, i, 0)),
            pl.BlockSpec((_BR,), lambda i: (i,)),
            pl.BlockSpec((H1, H2), lambda i: (0, 0)),
            pl.BlockSpec((H2,), lambda i: (0,)),
            pl.BlockSpec((H1, H2), lambda i: (0, 0)),
            pl.BlockSpec((H2,), lambda i: (0,)),
        ],
        out_specs=[
            pl.BlockSpec((_BR, H2), lambda i: (i, 0)),
            pl.BlockSpec((_BR, H2), lambda i: (i, 0)),
        ],
        out_shape=[
            jax.ShapeDtypeStruct((NP, H2), jnp.float32),
            jax.ShapeDtypeStruct((NP, H2), jnp.float32),
        ],
    )(a, dinv, wm, bm, ws, bs)


def kernel(x, edge_index, edge_attr, W1, b1, W2, b2, W_mu, b_mu, W_std, b_std):
    loop = jnp.arange(N, dtype=jnp.int32)
    pad = EP - E2
    src2 = jnp.concatenate([edge_index[0], loop, jnp.zeros((pad,), jnp.int32)])
    dst2 = jnp.concatenate([edge_index[1], loop, jnp.zeros((pad,), jnp.int32)])
    ew2 = jnp.concatenate(
        [edge_attr, jnp.ones((N,), jnp.float32), jnp.zeros((pad,), jnp.float32)])
    src2d = src2.reshape(EROWS, 128)
    dst2d = dst2.reshape(EROWS, 128)
    ew2d = ew2.reshape(EROWS, 128)
    x_pad = jnp.pad(x, ((0, NP - N), (0, 0)))

    degp = _deg_kernel(dst2d, ew2d)
    dinv, t0 = _t1(degp, x_pad)
    a1 = _agg(t0, src2d, dst2d, ew2d)
    t1 = _mid1(a1, dinv, W1, b1)
    a2 = _agg(t1, src2d, dst2d, ew2d)
    t2 = _mid2(a2, dinv, W2, b2)
    a3 = _agg(t2, src2d, dst2d, ew2d)
    mu_p, std_p = _head(a3, dinv, W_mu, b_mu, W_std, b_std)
    return (mu_p[:N], std_p[:N])
